# baseline (speedup 1.0000x reference)
import functools

import jax
import jax.numpy as jnp
from jax import lax
from jax.experimental import pallas as pl
from jax.experimental.pallas import tpu as pltpu
from jax.experimental.pallas import tpu_sc as plsc

_B = 4096
_N = 200
_D = 64
_K = 8
_L = 16
_NC = 2
_NS = 16
_NW = _NC * _NS
_BAGS_PER_W = _B // _NW
_BLK = _L
_NBLK = _BAGS_PER_W // _BLK
_NEG_INF = float("-inf")
_LN2 = 0.6931471805599453
_LNK = 2.0794415416798357


def _log_1_to_8(x):
    bits = lax.bitcast_convert_type(x, jnp.int32)
    e = lax.convert_element_type((bits >> 23) - 127, jnp.float32)
    m = lax.bitcast_convert_type(
        (bits & jnp.int32(0x007FFFFF)) | jnp.int32(0x3F800000), jnp.float32)
    z = (m - 1.0) / (m + 1.0)
    z2 = z * z
    p = 1.0 / 9.0 + z2 * (1.0 / 11.0)
    p = 1.0 / 7.0 + z2 * p
    p = 1.0 / 5.0 + z2 * p
    p = 1.0 / 3.0 + z2 * p
    p = 1.0 + z2 * p
    return e * _LN2 + 2.0 * z * p


@functools.partial(jax.jit, donate_argnums=())
def _sc_call(scores, mask_f, reprs_flat):
    mesh = plsc.VectorSubcoreMesh(
        core_axis_name="c", subcore_axis_name="s",
        num_cores=_NC, num_subcores=_NS)

    @functools.partial(
        pl.kernel,
        out_type=(
            jax.ShapeDtypeStruct((_B,), jnp.float32),
            jax.ShapeDtypeStruct((_B, _N), jnp.float32),
            jax.ShapeDtypeStruct((_B, _D), jnp.float32),
        ),
        mesh=mesh,
        compiler_params=pltpu.CompilerParams(use_tc_tiling_on_sc=False,
                                             needs_layout_passes=False),
        scratch_types=[
            pltpu.VMEM((_BLK, _N), jnp.float32),
            pltpu.VMEM((_BLK, _N), jnp.float32),
            pltpu.VMEM((_BLK, _N), jnp.float32),
            pltpu.VMEM((_BLK * _K,), jnp.int32),
            pltpu.VMEM((_BLK * _K, _D), jnp.float32),
            pltpu.VMEM((_BLK, _D), jnp.float32),
            pltpu.VMEM((_BAGS_PER_W,), jnp.float32),
            pltpu.SemaphoreType.DMA,
        ],
    )
    def sc_kernel(scores_hbm, mask_hbm, reprs_hbm,
                  agg_hbm, w_hbm, rep_hbm,
                  sc_v, mf_v, w_v, idx_v, rows_v, rep_v, agg_v,
                  gsem):
        wid = lax.axis_index("s") * _NC + lax.axis_index("c")
        lanes = lax.iota(jnp.int32, _L)
        zeros_i = jnp.zeros((_L,), jnp.int32)
        zeros_f = jnp.zeros((_L,), jnp.float32)

        def masked_col(j):
            jj = zeros_i + j
            s = plsc.load_gather(sc_v, [lanes, jj])
            mf = plsc.load_gather(mf_v, [lanes, jj])
            return jnp.where(mf > 0.0, s, _NEG_INF), jj

        def block_body(blk, _):
            row0 = wid * _BAGS_PER_W + blk * _BLK
            pltpu.sync_copy(scores_hbm.at[pl.ds(row0, _BLK), :], sc_v)
            pltpu.sync_copy(mask_hbm.at[pl.ds(row0, _BLK), :], mf_v)

            def p1_body(jo, t):
                t = list(t)
                for ju in range(8):
                    s, _ = masked_col(jo * 8 + ju)
                    c = s
                    for i in range(8):
                        hi = jnp.maximum(t[i], c)
                        c = jnp.minimum(t[i], c)
                        t[i] = hi
                return tuple(t)

            t = lax.fori_loop(0, _N // 8, p1_body,
                              tuple(zeros_f + _NEG_INF for _ in range(8)))

            big_t = t[0]
            thr = t[7]
            empty = big_t == _NEG_INF
            c_gt = zeros_i
            for i in range(7):
                c_gt = c_gt + jnp.where(t[i] > thr, 1, 0)
            budget = jnp.where(empty, 0, _K - c_gt)

            ssum = zeros_f
            for i in range(8):
                ssum = ssum + jnp.exp(t[i] - big_t)
            agg = big_t + _log_1_to_8(ssum) - _LNK
            agg = jnp.where(empty, 0.0, agg)
            plsc.store_scatter(agg_v, [blk * _BLK + lanes], agg)
            scale_vec = jnp.where(empty, 0.0, 1.0 / _K)

            for i in range(_BLK * _K // _L):
                idx_v[pl.ds(i * _L, _L)] = zeros_i

            gbase = (row0 + lanes) * _N

            def p2_body(jo, carry):
                eq_cnt, cnt = carry
                for ju in range(8):
                    j = jo * 8 + ju
                    s, jj = masked_col(j)
                    take_eq = (s == thr) & (eq_cnt < budget)
                    take = (s > thr) | take_eq
                    plsc.store_scatter(w_v, [lanes, jj],
                                      jnp.where(take, 1.0 / _K, 0.0))
                    plsc.store_scatter(idx_v, [lanes * _K + cnt],
                                      gbase + jj, mask=take)
                    eq_cnt = eq_cnt + jnp.where(take_eq, 1, 0)
                    cnt = cnt + jnp.where(take, 1, 0)
                return eq_cnt, cnt

            lax.fori_loop(0, _N // 8, p2_body, (zeros_i, zeros_i))

            pltpu.sync_copy(w_v, w_hbm.at[pl.ds(row0, _BLK), :])

            copies = []
            for b in range(_BLK):
                copies.append(pltpu.async_copy(
                    reprs_hbm.at[idx_v.at[pl.ds(b * _K, _K)]],
                    rows_v.at[pl.ds(b * _K, _K)], gsem))
            for cp in copies:
                cp.wait()

            for b in range(_BLK):
                scale = scale_vec[b]
                for c in range(_D // _L):
                    acc = rows_v[b * _K, pl.ds(c * _L, _L)]
                    for r in range(1, _K):
                        acc = acc + rows_v[b * _K + r, pl.ds(c * _L, _L)]
                    rep_v[b, pl.ds(c * _L, _L)] = acc * scale

            pltpu.sync_copy(rep_v, rep_hbm.at[pl.ds(row0, _BLK), :])
            return _

        lax.fori_loop(0, _NBLK, block_body, 0)
        pltpu.sync_copy(agg_v, agg_hbm.at[pl.ds(wid * _BAGS_PER_W,
                                                _BAGS_PER_W)])

    return sc_kernel(scores, mask_f, reprs_flat)


def kernel(pair_repr, path_scores, path_reprs, bag_mask, W, b):
    del pair_repr, W, b
    mask_f = bag_mask.astype(jnp.float32)
    reprs_flat = path_reprs.reshape(_B * _N, _D)
    agg_score, weights, agg_repr = _sc_call(path_scores, mask_f, reprs_flat)
    return (agg_score, weights, agg_repr)
